# baseline (speedup 1.0000x reference)
import functools

import jax
import jax.numpy as jnp
from jax import lax
from jax.experimental import pallas as pl
from jax.experimental.pallas import tpu as pltpu
from jax.experimental.pallas import tpu_sc as plsc

E = 320000
D = 128
NC, NS, L = 2, 16, 16
NW = NC * NS
E_W = E // NW
C = 200
N_CHUNK = E_W // C

_mesh = plsc.VectorSubcoreMesh(core_axis_name="c", subcore_axis_name="s")


@functools.partial(
    pl.kernel,
    mesh=_mesh,
    out_type=jax.ShapeDtypeStruct((E,), jnp.float32),
    scratch_types=[
        pltpu.VMEM((C,), jnp.int32),
        pltpu.VMEM((C,), jnp.int32),
        pltpu.VMEM((C, D // 2), jnp.int32),
        pltpu.VMEM((C, D // 2), jnp.int32),
        pltpu.VMEM((C + L,), jnp.float32),
        pltpu.SemaphoreType.DMA,
    ],
    compiler_params=pltpu.CompilerParams(needs_layout_passes=False,
                                         use_tc_tiling_on_sc=False),
)
def _edge_dot(drug, prot, idx0, idx1, out, idx0_v, idx1_v, r0, r1, o_v, sem):
    wid = lax.axis_index("s") * NC + lax.axis_index("c")
    base = wid * E_W
    last_lane = lax.iota(jnp.int32, L) == (L - 1)

    def chunk_body(c, carry):
        off = base + c * C
        pltpu.sync_copy(idx0.at[pl.ds(off, C)], idx0_v)
        pltpu.sync_copy(idx1.at[pl.ds(off, C)], idx1_v)
        cp0 = pltpu.async_copy(drug.at[idx0_v], r0, sem)
        cp1 = pltpu.async_copy(prot.at[idx1_v], r1, sem)
        cp0.wait()
        cp1.wait()

        def edge_body(e, carry2):
            acc = None
            for k in range(D // (2 * L)):
                a = plsc.bitcast(r0[e, pl.ds(k * L, L)], jnp.bfloat16)
                b = plsc.bitcast(r1[e, pl.ds(k * L, L)], jnp.bfloat16)
                p0, p1 = plsc.unpack(a * b,
                                     format=plsc.PackFormat.INTERLEAVED)
                acc = p0 + p1 if acc is None else acc + p0 + p1
            cs = plsc.cumsum(acc)
            plsc.store_compressed(o_v.at[pl.ds(e, L)], cs, mask=last_lane)
            return carry2

        lax.fori_loop(0, C, edge_body, 0)
        pltpu.sync_copy(o_v.at[pl.ds(0, C)], out.at[pl.ds(off, C)])
        return carry

    lax.fori_loop(0, N_CHUNK, chunk_body, 0)


def kernel(x_drug, x_prot, edge_label_index):
    idx = edge_label_index.astype(jnp.int32)
    drug_w = jax.lax.bitcast_convert_type(
        x_drug.astype(jnp.bfloat16).reshape(-1, D // 2, 2), jnp.int32)
    prot_w = jax.lax.bitcast_convert_type(
        x_prot.astype(jnp.bfloat16).reshape(-1, D // 2, 2), jnp.int32)
    return _edge_dot(drug_w, prot_w, idx[0], idx[1])


# speedup vs baseline: 2.5129x; 2.5129x over previous
import functools

import jax
import jax.numpy as jnp
from jax import lax
from jax.experimental import pallas as pl
from jax.experimental.pallas import tpu as pltpu
from jax.experimental.pallas import tpu_sc as plsc

E = 320000
D = 128
W = D // 2
NC, NS, L = 2, 16, 16
NW = NC * NS
E_W = E // NW
C = 200
N_CHUNK = E_W // C
UNROLL = 8

_mesh = plsc.VectorSubcoreMesh(core_axis_name="c", subcore_axis_name="s")


@functools.partial(
    pl.kernel,
    mesh=_mesh,
    out_type=jax.ShapeDtypeStruct((E,), jnp.float32),
    scratch_types=[
        pltpu.VMEM((E_W,), jnp.int32),
        pltpu.VMEM((E_W,), jnp.int32),
        pltpu.VMEM((2, C, W), jnp.int32),
        pltpu.VMEM((2, C, W), jnp.int32),
        pltpu.VMEM((2, C + L), jnp.float32),
        pltpu.SemaphoreType.DMA,
        pltpu.SemaphoreType.DMA,
        pltpu.SemaphoreType.DMA,
        pltpu.SemaphoreType.DMA,
        pltpu.SemaphoreType.DMA,
    ],
    compiler_params=pltpu.CompilerParams(needs_layout_passes=False,
                                         use_tc_tiling_on_sc=False),
)
def _edge_dot(drug, prot, idx0, idx1, out, idx0_v, idx1_v, r0, r1, o_v,
              sem_i, sem_g0, sem_g1, sem_o0, sem_o1):
    sem_g = (sem_g0, sem_g1)
    sem_o = (sem_o0, sem_o1)
    wid = lax.axis_index("s") * NC + lax.axis_index("c")
    base = wid * E_W
    last_lane = lax.iota(jnp.int32, L) == (L - 1)

    ci0 = pltpu.async_copy(idx0.at[pl.ds(base, E_W)], idx0_v, sem_i)
    ci1 = pltpu.async_copy(idx1.at[pl.ds(base, E_W)], idx1_v, sem_i)
    ci0.wait()
    ci1.wait()

    def issue_gather(c, buf):
        pltpu.async_copy(drug.at[idx0_v.at[pl.ds(c * C, C)]], r0.at[buf],
                         sem_g[buf])
        pltpu.async_copy(prot.at[idx1_v.at[pl.ds(c * C, C)]], r1.at[buf],
                         sem_g[buf])

    def wait_gather(buf):
        pltpu.make_async_copy(drug.at[idx0_v.at[pl.ds(0, C)]], r0.at[buf],
                              sem_g[buf]).wait()
        pltpu.make_async_copy(prot.at[idx1_v.at[pl.ds(0, C)]], r1.at[buf],
                              sem_g[buf]).wait()

    def wait_out(c, buf):
        pltpu.make_async_copy(o_v.at[buf, pl.ds(0, C)],
                              out.at[pl.ds(base + c * C, C)],
                              sem_o[buf]).wait()

    issue_gather(0, 0)

    def outer(c0, carry):
        for b in range(2):
            c = c0 + b
            nb = 1 - b

            @pl.when(c + 1 < N_CHUNK)
            def _():
                issue_gather(c + 1, nb)

            @pl.when(c >= 2)
            def _():
                wait_out(c - 2, b)

            wait_gather(b)

            @plsc.parallel_loop(0, C, unroll=UNROLL)
            def edge_body(e):
                acc = None
                for k in range(W // L):
                    a = plsc.bitcast(r0[b, e, pl.ds(k * L, L)], jnp.bfloat16)
                    v = plsc.bitcast(r1[b, e, pl.ds(k * L, L)], jnp.bfloat16)
                    p0, p1 = plsc.unpack(a * v,
                                         format=plsc.PackFormat.INTERLEAVED)
                    acc = p0 + p1 if acc is None else acc + p0 + p1
                cs = plsc.cumsum(acc)
                plsc.store_compressed(o_v.at[b, pl.ds(e, L)], cs,
                                      mask=last_lane)

            pltpu.async_copy(o_v.at[b, pl.ds(0, C)],
                             out.at[pl.ds(base + c * C, C)], sem_o[b])
        return carry

    lax.fori_loop(0, N_CHUNK // 2, lambda i, carry: outer(i * 2, carry), 0)
    wait_out(N_CHUNK - 2, 0)
    wait_out(N_CHUNK - 1, 1)


def kernel(x_drug, x_prot, edge_label_index):
    idx = edge_label_index.astype(jnp.int32)
    drug_w = jax.lax.bitcast_convert_type(
        x_drug.astype(jnp.bfloat16).reshape(-1, W, 2), jnp.int32)
    prot_w = jax.lax.bitcast_convert_type(
        x_prot.astype(jnp.bfloat16).reshape(-1, W, 2), jnp.int32)
    return _edge_dot(drug_w, prot_w, idx[0], idx[1])
